# baseline (speedup 1.0000x reference)
import functools

import jax
import jax.numpy as jnp
from jax import lax
from jax.experimental import pallas as pl
from jax.experimental.pallas import tpu as pltpu
from jax.experimental.pallas import tpu_sc as plsc

N = 10000
E = 320000
D_IN = 128
H = 512
D_OUT = 128

NC = 2
NS = 16
NW = NC * NS

K = 128
CH = 80
EPT = CH * K
E_PAD = NW * EPT
NP = 10240
RS = NP // NS
SK = 128
RB = RS // SK

_MESH = plsc.VectorSubcoreMesh(
    core_axis_name="c", subcore_axis_name="s", num_cores=NC, num_subcores=NS
)

_F32 = jnp.float32


def _deg_body(dstm, ones_h, z1, degp, acc, dst_all, ones_v, stage, sem):
    del sem
    cid = lax.axis_index("c")
    sid = lax.axis_index("s")
    wid = sid * NC + cid
    off = pl.multiple_of(sid * RS, 8)
    pltpu.sync_copy(dstm.at[wid], dst_all)
    pltpu.sync_copy(ones_h, ones_v)
    pltpu.sync_copy(z1, stage)
    pltpu.sync_copy(stage, acc.at[pl.ds(off, RS)])
    plsc.subcore_barrier()

    @pl.loop(0, CH)
    def _chunk(c):
        pltpu.sync_copy(ones_v, acc.at[dst_all.at[c]], add=True)

    plsc.subcore_barrier()
    pltpu.sync_copy(acc.at[pl.ds(off, RS)], stage)
    pltpu.sync_copy(stage,
                    degp.at[pl.ds(pl.multiple_of(cid * NP + off, 8), RS)])


_DEG = pl.kernel(
    _deg_body,
    out_type=jax.ShapeDtypeStruct((NC * NP,), _F32),
    mesh=_MESH,
    scratch_types=[
        pltpu.VMEM_SHARED((NP,), _F32),
        pltpu.VMEM((CH, K), jnp.int32),
        pltpu.VMEM((K,), _F32),
        pltpu.VMEM((RS,), _F32),
        pltpu.SemaphoreType.DMA,
    ],
)


def _agg_body(nblk, *refs):
    tabs = refs[:nblk]
    srcm, dstm, z2 = refs[nblk:nblk + 3]
    outs = refs[nblk + 3:2 * nblk + 3]
    acc, src_all, dst_all, r0, sem0 = refs[2 * nblk + 3:]
    cid = lax.axis_index("c")
    sid = lax.axis_index("s")
    wid = sid * NC + cid
    off = pl.multiple_of(sid * RS, 8)
    pltpu.sync_copy(srcm.at[wid], src_all)
    pltpu.sync_copy(dstm.at[wid], dst_all)
    for b in range(nblk):
        tab = tabs[b]
        pltpu.sync_copy(z2, r0.at[pl.ds(0, SK)])
        for j in range(RB):
            pltpu.sync_copy(r0.at[pl.ds(0, SK)],
                            acc.at[pl.ds(pl.multiple_of(off + j * SK, 8), SK)])
        plsc.subcore_barrier()

        @pl.loop(0, CH)
        def _chunk(c):
            pltpu.async_copy(tab.at[src_all.at[c]], r0, sem0).wait()
            pltpu.sync_copy(r0, acc.at[dst_all.at[c]], add=True)

        plsc.subcore_barrier()
        for j in range(RB):
            joff = pl.multiple_of(off + j * SK, 8)
            pltpu.sync_copy(acc.at[pl.ds(joff, SK)], r0.at[pl.ds(0, SK)])
            pltpu.sync_copy(r0.at[pl.ds(0, SK)],
                            outs[b].at[cid, pl.ds(joff, SK)])


def _make_agg(nblk):
    return pl.kernel(
        functools.partial(_agg_body, nblk),
        out_type=[jax.ShapeDtypeStruct((NC, NP, 128), _F32)] * nblk,
        mesh=_MESH,
        scratch_types=[
            pltpu.VMEM_SHARED((NP, 128), _F32),
            pltpu.VMEM((CH, K), jnp.int32),
            pltpu.VMEM((CH, K), jnp.int32),
            pltpu.VMEM((K, 128), _F32),
            pltpu.SemaphoreType.DMA,
        ],
    )


_AGG1 = _make_agg(1)
_AGG4 = _make_agg(4)


_GRID = NP // K


def _dinv_block(degp_t_ref):
    deg = degp_t_ref[:, 0:1] + degp_t_ref[:, 1:2] + 1.0
    return lax.rsqrt(deg)


def _tc1_body(degp_t, x, o):
    o[...] = x[...] * _dinv_block(degp_t)


def _tc2_body(degp_t, p, xs, w1, b1, o0, o1, o2, o3):
    dinv = _dinv_block(degp_t)
    y1 = (p[0] + p[1] + xs[...]) * dinv
    h1 = jax.nn.relu(
        jnp.dot(y1, w1[...], preferred_element_type=_F32) + b1[...])
    hs = h1 * dinv
    o0[...] = hs[:, 0:128]
    o1[...] = hs[:, 128:256]
    o2[...] = hs[:, 256:384]
    o3[...] = hs[:, 384:512]


def _tc3_body(degp_t, q0, q1, q2, q3, t0, t1, t2, t3, w2, b2, w3, o):
    dinv = _dinv_block(degp_t)
    y2 = jnp.concatenate(
        [(q0[0] + q0[1] + t0[...]) * dinv,
         (q1[0] + q1[1] + t1[...]) * dinv,
         (q2[0] + q2[1] + t2[...]) * dinv,
         (q3[0] + q3[1] + t3[...]) * dinv], axis=1)
    h2 = jax.nn.relu(
        jnp.dot(y2, w2[...], preferred_element_type=_F32) + b2[...])
    g = jnp.dot(h2, w3[...], preferred_element_type=_F32)
    o[...] = g * dinv


def _tc4_body(degp_t, p, gs, b3, o):
    dinv = _dinv_block(degp_t)
    o[...] = (p[0] + p[1] + gs[...]) * dinv + b3[...]


def _row_spec(shape_prefix=()):
    nd = len(shape_prefix)
    return pl.BlockSpec(shape_prefix + (K, 128),
                        lambda i, nd=nd: (0,) * nd + (i, 0))


_DEGP_SPEC = pl.BlockSpec((K, NC), lambda i: (i, 0))


def _full_spec(shape):
    nd = len(shape)
    return pl.BlockSpec(shape, lambda i, nd=nd: (0,) * nd)


def _tc1(degp_t, x_pad):
    return pl.pallas_call(
        _tc1_body,
        grid=(_GRID,),
        in_specs=[_DEGP_SPEC, _row_spec()],
        out_specs=_row_spec(),
        out_shape=jax.ShapeDtypeStruct((NP, 128), _F32),
    )(degp_t, x_pad)


def _tc2(degp_t, p1, xs, w1, b1):
    return pl.pallas_call(
        _tc2_body,
        grid=(_GRID,),
        in_specs=[_DEGP_SPEC, _row_spec((NC,)), _row_spec(),
                  _full_spec((D_IN, H)), _full_spec((1, H))],
        out_specs=[_row_spec()] * 4,
        out_shape=[jax.ShapeDtypeStruct((NP, 128), _F32)] * 4,
    )(degp_t, p1, xs, w1, b1)


def _tc3(degp_t, p2, h1s4, w2, b2, w3):
    return pl.pallas_call(
        _tc3_body,
        grid=(_GRID,),
        in_specs=[_DEGP_SPEC] + [_row_spec((NC,))] * 4 + [_row_spec()] * 4 +
                 [_full_spec((H, H)), _full_spec((1, H)),
                  _full_spec((H, D_OUT))],
        out_specs=_row_spec(),
        out_shape=jax.ShapeDtypeStruct((NP, 128), _F32),
    )(degp_t, *p2, *h1s4, w2, b2, w3)


def _tc4(degp_t, p3, gs, b3):
    return pl.pallas_call(
        _tc4_body,
        grid=((N + K - 1) // K,),
        in_specs=[_DEGP_SPEC, _row_spec((NC,)), _row_spec(),
                  _full_spec((1, D_OUT))],
        out_specs=_row_spec(),
        out_shape=jax.ShapeDtypeStruct((N, 128), _F32),
    )(degp_t, p3, gs, b3)


def kernel(x, edge_index, W1, b1, W2, b2, W3, b3):
    src = edge_index[0]
    dst = edge_index[1]
    pad_idx = jnp.full((E_PAD - E,), N, jnp.int32)
    srcm = jnp.concatenate([src, pad_idx]).reshape(NW, CH, K)
    dstm = jnp.concatenate([dst, pad_idx]).reshape(NW, CH, K)
    ones_k = jnp.ones((K,), _F32)
    z1 = jnp.zeros((RS,), _F32)
    z2 = jnp.zeros((SK, 128), _F32)
    x_pad = jnp.pad(x, ((0, NP - N), (0, 0)))

    degp = _DEG(dstm, ones_k, z1)
    degp_t = degp.reshape(NC, NP).T

    xs = _tc1(degp_t, x_pad)
    p1 = _AGG1(xs, srcm, dstm, z2)[0]
    h1s4 = _tc2(degp_t, p1, xs, W1, b1.reshape(1, H))
    p2 = _AGG4(*h1s4, srcm, dstm, z2)
    gs = _tc3(degp_t, p2, h1s4, W2, b2.reshape(1, H), W3)
    p3 = _AGG1(gs, srcm, dstm, z2)[0]
    return _tc4(degp_t, p3, gs, b3.reshape(1, D_OUT))


# speedup vs baseline: 1.5532x; 1.5532x over previous
import functools

import jax
import jax.numpy as jnp
from jax import lax
from jax.experimental import pallas as pl
from jax.experimental.pallas import tpu as pltpu
from jax.experimental.pallas import tpu_sc as plsc

N = 10000
E = 320000
D_IN = 128
H = 512
D_OUT = 128

NC = 2
NS = 16
NW = NC * NS

K = 128
CH = 79
EPT = CH * K
E_PAD = NW * EPT
NP = 10240
RS = NP // NS
SK = 128
RB = RS // SK

_MESH = plsc.VectorSubcoreMesh(
    core_axis_name="c", subcore_axis_name="s", num_cores=NC, num_subcores=NS
)

_F32 = jnp.float32


def _deg_body(dstm, ones_h, z1, degp, acc, dst_all, ones_v, stage, sem):
    del sem
    cid = lax.axis_index("c")
    sid = lax.axis_index("s")
    wid = sid * NC + cid
    off = pl.multiple_of(sid * RS, 8)
    pltpu.sync_copy(dstm.at[wid], dst_all)
    pltpu.sync_copy(ones_h, ones_v)
    pltpu.sync_copy(z1, stage)
    pltpu.sync_copy(stage, acc.at[pl.ds(off, RS)])
    plsc.subcore_barrier()

    @pl.loop(0, CH)
    def _chunk(c):
        pltpu.sync_copy(ones_v, acc.at[dst_all.at[c]], add=True)

    plsc.subcore_barrier()
    pltpu.sync_copy(acc.at[pl.ds(off, RS)], stage)
    pltpu.sync_copy(stage,
                    degp.at[pl.ds(pl.multiple_of(cid * NP + off, 8), RS)])


_DEG = pl.kernel(
    _deg_body,
    out_type=jax.ShapeDtypeStruct((NC * NP,), _F32),
    mesh=_MESH,
    scratch_types=[
        pltpu.VMEM_SHARED((NP,), _F32),
        pltpu.VMEM((CH, K), jnp.int32),
        pltpu.VMEM((K,), _F32),
        pltpu.VMEM((RS,), _F32),
        pltpu.SemaphoreType.DMA,
    ],
)


def _agg_body(nblk, *refs):
    tabs = refs[:nblk]
    srcm, dstm, z2 = refs[nblk:nblk + 3]
    outs = refs[nblk + 3:2 * nblk + 3]
    acc, src_all, dst_all, r0, sem0 = refs[2 * nblk + 3:]
    cid = lax.axis_index("c")
    sid = lax.axis_index("s")
    wid = sid * NC + cid
    off = pl.multiple_of(sid * RS, 8)
    pltpu.sync_copy(srcm.at[wid], src_all)
    pltpu.sync_copy(dstm.at[wid], dst_all)
    for b in range(nblk):
        tab = tabs[b]
        pltpu.sync_copy(z2, r0)
        for j in range(RB):
            pltpu.sync_copy(
                r0, acc.at[pl.ds(pl.multiple_of(off + j * SK, 8), SK)])
        plsc.subcore_barrier()

        @pl.loop(0, CH)
        def _chunk(c):
            pltpu.async_copy(tab.at[src_all.at[c]], r0, sem0).wait()
            pltpu.sync_copy(r0, acc.at[dst_all.at[c]], add=True)

        plsc.subcore_barrier()
        for j in range(RB):
            joff = pl.multiple_of(off + j * SK, 8)
            pltpu.sync_copy(acc.at[pl.ds(joff, SK)], r0)
            pltpu.sync_copy(r0, outs[b].at[cid, pl.ds(joff, SK)])


def _make_agg(nblk):
    return pl.kernel(
        functools.partial(_agg_body, nblk),
        out_type=[jax.ShapeDtypeStruct((NC, NP, 128), _F32)] * nblk,
        mesh=_MESH,
        scratch_types=[
            pltpu.VMEM_SHARED((NP, 128), _F32),
            pltpu.VMEM((CH, K), jnp.int32),
            pltpu.VMEM((CH, K), jnp.int32),
            pltpu.VMEM((K, 128), _F32),
            pltpu.SemaphoreType.DMA,
        ],
    )


_AGG1 = _make_agg(1)
_AGG4 = _make_agg(4)


_GRID = NP // K


def _dinv_block(degp_t_ref):
    deg = degp_t_ref[:, 0:1] + degp_t_ref[:, 1:2] + 1.0
    return lax.rsqrt(deg)


def _tc1_body(degp_t, x, o):
    o[...] = x[...] * _dinv_block(degp_t)


def _tc2_body(degp_t, p, xs, w1, b1, o0, o1, o2, o3):
    dinv = _dinv_block(degp_t)
    y1 = (p[0] + p[1] + xs[...]) * dinv
    h1 = jax.nn.relu(
        jnp.dot(y1, w1[...], preferred_element_type=_F32) + b1[...])
    hs = h1 * dinv
    o0[...] = hs[:, 0:128]
    o1[...] = hs[:, 128:256]
    o2[...] = hs[:, 256:384]
    o3[...] = hs[:, 384:512]


def _tc3_body(degp_t, q0, q1, q2, q3, t0, t1, t2, t3, w2, b2, w3, o):
    dinv = _dinv_block(degp_t)
    y2 = jnp.concatenate(
        [(q0[0] + q0[1] + t0[...]) * dinv,
         (q1[0] + q1[1] + t1[...]) * dinv,
         (q2[0] + q2[1] + t2[...]) * dinv,
         (q3[0] + q3[1] + t3[...]) * dinv], axis=1)
    h2 = jax.nn.relu(
        jnp.dot(y2, w2[...], preferred_element_type=_F32) + b2[...])
    g = jnp.dot(h2, w3[...], preferred_element_type=_F32)
    o[...] = g * dinv


def _tc4_body(degp_t, p, gs, b3, o):
    dinv = _dinv_block(degp_t)
    o[...] = (p[0] + p[1] + gs[...]) * dinv + b3[...]


def _row_spec(shape_prefix=()):
    nd = len(shape_prefix)
    return pl.BlockSpec(shape_prefix + (K, 128),
                        lambda i, nd=nd: (0,) * nd + (i, 0))


_DEGP_SPEC = pl.BlockSpec((K, NC), lambda i: (i, 0))


def _full_spec(shape):
    nd = len(shape)
    return pl.BlockSpec(shape, lambda i, nd=nd: (0,) * nd)


def _tc1(degp_t, x_pad):
    return pl.pallas_call(
        _tc1_body,
        grid=(_GRID,),
        in_specs=[_DEGP_SPEC, _row_spec()],
        out_specs=_row_spec(),
        out_shape=jax.ShapeDtypeStruct((NP, 128), _F32),
    )(degp_t, x_pad)


def _tc2(degp_t, p1, xs, w1, b1):
    return pl.pallas_call(
        _tc2_body,
        grid=(_GRID,),
        in_specs=[_DEGP_SPEC, _row_spec((NC,)), _row_spec(),
                  _full_spec((D_IN, H)), _full_spec((1, H))],
        out_specs=[_row_spec()] * 4,
        out_shape=[jax.ShapeDtypeStruct((NP, 128), _F32)] * 4,
    )(degp_t, p1, xs, w1, b1)


def _tc3(degp_t, p2, h1s4, w2, b2, w3):
    return pl.pallas_call(
        _tc3_body,
        grid=(_GRID,),
        in_specs=[_DEGP_SPEC] + [_row_spec((NC,))] * 4 + [_row_spec()] * 4 +
                 [_full_spec((H, H)), _full_spec((1, H)),
                  _full_spec((H, D_OUT))],
        out_specs=_row_spec(),
        out_shape=jax.ShapeDtypeStruct((NP, 128), _F32),
    )(degp_t, *p2, *h1s4, w2, b2, w3)


def _tc4(degp_t, p3, gs, b3):
    return pl.pallas_call(
        _tc4_body,
        grid=((N + K - 1) // K,),
        in_specs=[_DEGP_SPEC, _row_spec((NC,)), _row_spec(),
                  _full_spec((1, D_OUT))],
        out_specs=_row_spec(),
        out_shape=jax.ShapeDtypeStruct((N, 128), _F32),
    )(degp_t, p3, gs, b3)


def kernel(x, edge_index, W1, b1, W2, b2, W3, b3):
    src = edge_index[0]
    dst = edge_index[1]
    pad_idx = jnp.full((E_PAD - E,), N, jnp.int32)
    srcm = jnp.concatenate([src, pad_idx]).reshape(NW, CH, K)
    dstm = jnp.concatenate([dst, pad_idx]).reshape(NW, CH, K)
    ones_k = jnp.ones((K,), _F32)
    z1 = jnp.zeros((RS,), _F32)
    z2 = jnp.zeros((SK, 128), _F32)
    x_pad = jnp.pad(x, ((0, NP - N), (0, 0)))

    degp = _DEG(dstm, ones_k, z1)
    degp_t = degp.reshape(NC, NP).T

    xs = _tc1(degp_t, x_pad)
    p1 = _AGG1(xs, srcm, dstm, z2)[0]
    h1s4 = _tc2(degp_t, p1, xs, W1, b1.reshape(1, H))
    p2 = _AGG4(*h1s4, srcm, dstm, z2)
    gs = _tc3(degp_t, p2, h1s4, W2, b2.reshape(1, H), W3)
    p3 = _AGG1(gs, srcm, dstm, z2)[0]
    return _tc4(degp_t, p3, gs, b3.reshape(1, D_OUT))
